# Initial kernel scaffold; baseline (speedup 1.0000x reference)
import jax
import jax.numpy as jnp
from jax.experimental import pallas as pl


def kernel(x, edge_index, edge_weights, batch, W_in, b_in, W1, b1, W2, b2, Wc, bc):
    raise NotImplementedError("write your pallas kernel here")


# speedup vs baseline: 8.4425x; 8.4425x over previous
import functools

import jax
import jax.numpy as jnp
from jax import lax
from jax.experimental import pallas as pl
from jax.experimental.pallas import tpu as pltpu
from jax.experimental.pallas import tpu_sc as plsc

N = 10000
E = 320000
D = 128
NG = 64
NCLS = 10

NCORES = 2
NSUB = 16
NW = NCORES * NSUB
EPT = E // NW
CH = 80
NCHUNK = EPT // CH
DUMP = 80
NDCH = N // DUMP
DCPT = -(-NDCH // NSUB)

def _deg_body(dst_hbm, w_hbm, out_hbm, dst_v, wc_v, stage_v, page_v, acc_sh, sem):
    cid = lax.axis_index("c")
    sid = lax.axis_index("s")
    wid = sid * NCORES + cid

    zero16 = jnp.zeros((16,), jnp.float32)

    @pl.loop(0, DUMP)
    def _(i):
        page_v[i, :] = zero16

    for k in range(DCPT):
        j = k * NSUB + sid

        @pl.when(j < NDCH)
        def _():
            pltpu.sync_copy(page_v, acc_sh.at[pl.ds(j * DUMP, DUMP)])

    plsc.subcore_barrier()

    pltpu.sync_copy(dst_hbm.at[wid], dst_v)

    zero16i = jnp.zeros((16,), jnp.int32)

    @pl.loop(0, NCHUNK)
    def _(c):
        pltpu.sync_copy(w_hbm.at[wid, c], wc_v)

        @pl.loop(0, CH)
        def _(r):
            wbc = plsc.load_gather(wc_v, [zero16i, jnp.full((16,), r, jnp.int32)])
            stage_v[r, :] = wbc

        pltpu.sync_copy(stage_v, acc_sh.at[dst_v.at[c]], add=True)

    plsc.subcore_barrier()

    for k in range(DCPT):
        j = k * NSUB + sid

        @pl.when(j < NDCH)
        def _():
            pltpu.sync_copy(acc_sh.at[pl.ds(j * DUMP, DUMP)], page_v)
            pltpu.sync_copy(page_v, out_hbm.at[cid, pl.ds(j * DUMP, DUMP)])


def _edge_body(y_hbm, src_hbm, dst_hbm, w_hbm, out_hbm,
               src_v, dst_v, wc_v, rows_v, acc_sh, sem):
    page_v = rows_v
    cid = lax.axis_index("c")
    sid = lax.axis_index("s")
    wid = sid * NCORES + cid

    zero16 = jnp.zeros((16,), jnp.float32)

    @pl.loop(0, DUMP)
    def _(i):
        for j in range(D // 16):
            page_v[i, pl.ds(j * 16, 16)] = zero16

    for k in range(DCPT):
        j = k * NSUB + sid

        @pl.when(j < NDCH)
        def _():
            pltpu.sync_copy(page_v, acc_sh.at[pl.ds(j * DUMP, DUMP)])

    plsc.subcore_barrier()

    pltpu.sync_copy(src_hbm.at[wid], src_v)
    pltpu.sync_copy(dst_hbm.at[wid], dst_v)

    zero16i = jnp.zeros((16,), jnp.int32)

    @pl.loop(0, NCHUNK)
    def _(c):
        pltpu.sync_copy(w_hbm.at[wid, c], wc_v)
        pltpu.async_copy(y_hbm.at[src_v.at[c]], rows_v, sem).wait()

        @pl.loop(0, CH)
        def _(r):
            wbc = plsc.load_gather(wc_v, [zero16i, jnp.full((16,), r, jnp.int32)])
            for j in range(D // 16):
                rows_v[r, pl.ds(j * 16, 16)] = rows_v[r, pl.ds(j * 16, 16)] * wbc

        pltpu.sync_copy(rows_v, acc_sh.at[dst_v.at[c]], add=True)

    plsc.subcore_barrier()

    for k in range(DCPT):
        j = k * NSUB + sid

        @pl.when(j < NDCH)
        def _():
            pltpu.sync_copy(acc_sh.at[pl.ds(j * DUMP, DUMP)], page_v)
            pltpu.sync_copy(page_v, out_hbm.at[cid, pl.ds(j * DUMP, DUMP)])


@functools.lru_cache(maxsize=None)
def _sc_kernels():
    mesh = plsc.VectorSubcoreMesh(core_axis_name="c", subcore_axis_name="s")
    params = pltpu.CompilerParams(needs_layout_passes=False)
    deg = pl.kernel(
        _deg_body,
        out_type=jax.ShapeDtypeStruct((NCORES, N, 16), jnp.float32),
        mesh=mesh,
        compiler_params=params,
        scratch_types=[
            pltpu.VMEM((NCHUNK, CH), jnp.int32),
            pltpu.VMEM((1, CH), jnp.float32),
            pltpu.VMEM((CH, 16), jnp.float32),
            pltpu.VMEM((DUMP, 16), jnp.float32),
            pltpu.VMEM_SHARED((N, 16), jnp.float32),
            pltpu.SemaphoreType.DMA,
        ],
    )
    edge = pl.kernel(
        _edge_body,
        out_type=jax.ShapeDtypeStruct((NCORES, N, D), jnp.float32),
        mesh=mesh,
        compiler_params=params,
        scratch_types=[
            pltpu.VMEM((NCHUNK, CH), jnp.int32),
            pltpu.VMEM((NCHUNK, CH), jnp.int32),
            pltpu.VMEM((1, CH), jnp.float32),
            pltpu.VMEM((CH, D), jnp.float32),
            pltpu.VMEM_SHARED((N, D), jnp.float32),
            pltpu.SemaphoreType.DMA,
        ],
    )
    return deg, edge


def _deg_kernel(dst3, w4):
    return _sc_kernels()[0](dst3, w4)


def _edge_kernel(y, src3, dst3, w4):
    return _sc_kernels()[1](y, src3, dst3, w4)


RB = 1000
GRID = N // RB


def _tc1_body(x_ref, win_ref, bin_ref, w1_ref, dg0_ref, dg1_ref, y_ref, dis_ref):
    deg = dg0_ref[...] + dg1_ref[...] + 1.0
    dis = lax.rsqrt(deg)
    dis_ref[...] = dis
    h = jnp.maximum(jnp.dot(x_ref[...], win_ref[...],
                            preferred_element_type=jnp.float32) + bin_ref[...], 0.0)
    y_ref[...] = dis * jnp.dot(h, w1_ref[...], preferred_element_type=jnp.float32)


def _tc1(x, W_in, b_in, W1, dg0, dg1):
    return pl.pallas_call(
        _tc1_body,
        grid=(GRID,),
        in_specs=[
            pl.BlockSpec((RB, D), lambda i: (i, 0)),
            pl.BlockSpec((D, D), lambda i: (0, 0)),
            pl.BlockSpec((1, D), lambda i: (0, 0)),
            pl.BlockSpec((D, D), lambda i: (0, 0)),
            pl.BlockSpec((RB, 1), lambda i: (i, 0)),
            pl.BlockSpec((RB, 1), lambda i: (i, 0)),
        ],
        out_specs=[
            pl.BlockSpec((RB, D), lambda i: (i, 0)),
            pl.BlockSpec((RB, 1), lambda i: (i, 0)),
        ],
        out_shape=[
            jax.ShapeDtypeStruct((N, D), jnp.float32),
            jax.ShapeDtypeStruct((N, 1), jnp.float32),
        ],
    )(x, W_in, b_in, W1, dg0, dg1)


def _tc2_body(p0_ref, p1_ref, y_ref, dis_ref, b_ref, w_ref, out_ref):
    dis = dis_ref[...]
    h = jnp.maximum(dis * (p0_ref[...] + p1_ref[...] + y_ref[...]) + b_ref[...], 0.0)
    out_ref[...] = dis * jnp.dot(h, w_ref[...], preferred_element_type=jnp.float32)


def _tc2(p0, p1, y, dis, b, W):
    return pl.pallas_call(
        _tc2_body,
        grid=(GRID,),
        in_specs=[
            pl.BlockSpec((RB, D), lambda i: (i, 0)),
            pl.BlockSpec((RB, D), lambda i: (i, 0)),
            pl.BlockSpec((RB, D), lambda i: (i, 0)),
            pl.BlockSpec((RB, 1), lambda i: (i, 0)),
            pl.BlockSpec((1, D), lambda i: (0, 0)),
            pl.BlockSpec((D, D), lambda i: (0, 0)),
        ],
        out_specs=pl.BlockSpec((RB, D), lambda i: (i, 0)),
        out_shape=jax.ShapeDtypeStruct((N, D), jnp.float32),
    )(p0, p1, y, dis, b, W)


def _tc3_body(p0_ref, p1_ref, y_ref, dis_ref, b_ref, batch_ref, wc_ref, bc_ref,
              out_ref, sums_ref, cnts_ref):
    i = pl.program_id(0)

    @pl.when(i == 0)
    def _():
        sums_ref[...] = jnp.zeros_like(sums_ref)
        cnts_ref[...] = jnp.zeros_like(cnts_ref)

    dis = dis_ref[...]
    h = jnp.maximum(dis * (p0_ref[...] + p1_ref[...] + y_ref[...]) + b_ref[...], 0.0)
    b = batch_ref[...]
    iota = lax.broadcasted_iota(jnp.int32, (RB, NG), 1)
    onehot = (iota == b).astype(jnp.float32)
    dn = (((0,), (0,)), ((), ()))
    sums_ref[...] += lax.dot_general(onehot, h, dn,
                                     preferred_element_type=jnp.float32)
    cnts_ref[...] += lax.dot_general(onehot, jnp.ones((RB, 1), jnp.float32), dn,
                                     preferred_element_type=jnp.float32)

    @pl.when(i == GRID - 1)
    def _():
        rep = sums_ref[...] / jnp.maximum(cnts_ref[...], 1.0)
        out_ref[...] = jnp.dot(rep, wc_ref[...],
                               preferred_element_type=jnp.float32) + bc_ref[...]


def _tc3(p0, p1, y, dis, b, batch2, Wc, bc):
    return pl.pallas_call(
        _tc3_body,
        grid=(GRID,),
        in_specs=[
            pl.BlockSpec((RB, D), lambda i: (i, 0)),
            pl.BlockSpec((RB, D), lambda i: (i, 0)),
            pl.BlockSpec((RB, D), lambda i: (i, 0)),
            pl.BlockSpec((RB, 1), lambda i: (i, 0)),
            pl.BlockSpec((1, D), lambda i: (0, 0)),
            pl.BlockSpec((RB, 1), lambda i: (i, 0)),
            pl.BlockSpec((D, NCLS), lambda i: (0, 0)),
            pl.BlockSpec((1, NCLS), lambda i: (0, 0)),
        ],
        out_specs=pl.BlockSpec((NG, NCLS), lambda i: (0, 0)),
        out_shape=jax.ShapeDtypeStruct((NG, NCLS), jnp.float32),
        scratch_shapes=[
            pltpu.VMEM((NG, D), jnp.float32),
            pltpu.VMEM((NG, 1), jnp.float32),
        ],
    )(p0, p1, y, dis, b, batch2, Wc, bc)


def kernel(x, edge_index, edge_weights, batch, W_in, b_in, W1, b1, W2, b2, Wc, bc):
    src3 = edge_index[0].astype(jnp.int32).reshape(NW, NCHUNK, CH)
    dst3 = edge_index[1].astype(jnp.int32).reshape(NW, NCHUNK, CH)
    w4 = edge_weights.astype(jnp.float32).reshape(NW, NCHUNK, 1, CH)

    ones_t = jnp.ones((N, D), jnp.float32)
    deg_parts = _edge_kernel(ones_t, src3, dst3, w4)
    dg0 = lax.slice(deg_parts, (0, 0, 0), (1, N, 1)).reshape(N, 1)
    dg1 = lax.slice(deg_parts, (1, 0, 0), (2, N, 1)).reshape(N, 1)

    y1, dis = _tc1(x, W_in, b_in.reshape(1, D), W1, dg0, dg1)

    p1 = _edge_kernel(y1, src3, dst3, w4)
    y2 = _tc2(p1[0], p1[1], y1, dis, b1.reshape(1, D), W2)

    p2 = _edge_kernel(y2, src3, dst3, w4)
    logits = _tc3(p2[0], p2[1], y2, dis, b2.reshape(1, D),
                  batch.astype(jnp.int32).reshape(N, 1), Wc, bc.reshape(1, NCLS))
    return logits
